# baseline (speedup 1.0000x reference)
import functools

import jax
import jax.numpy as jnp
from jax import lax
from jax.experimental import pallas as pl
from jax.experimental.pallas import tpu as pltpu
from jax.experimental.pallas import tpu_sc as plsc

N = 10000
NUM_RNA = 5000
E = 320000
D = 128

NC = 2
NS = 16
NW = NC * NS
CHUNK = 128
C0 = 98
C1 = 62
R0 = 12250
R1 = 7750
EP = NS * CHUNK * (C0 + C1)
NCHUNKS = EP // CHUNK
TRASH = N
NP = 10240
ZROWS = NP // NS


def _make_sc_scatter(with_cnt):
    mesh = plsc.VectorSubcoreMesh(core_axis_name="c", subcore_axis_name="s")
    out_type = [jax.ShapeDtypeStruct((NC, NP, D), jnp.float32)]
    scratch = [
        pltpu.VMEM_SHARED((NP, D), jnp.float32),
        pltpu.VMEM((2, CHUNK), jnp.int32),
        pltpu.VMEM((CHUNK, D), jnp.float32),
        pltpu.SemaphoreType.DMA,
    ]

    def body_nocnt(x_hbm, sd_hbm, zsum_hbm,
                   sum_out, acc_sum, sd, rows, sem):
        c = lax.axis_index("c")
        s = lax.axis_index("s")
        is0 = c == 0
        nch = jnp.where(is0, C0, C1)
        chbase = jnp.where(is0, s * C0, NS * C0 + s * C1)
        pltpu.sync_copy(zsum_hbm, acc_sum.at[pl.ds(s * ZROWS, ZROWS)])
        plsc.subcore_barrier()

        def chunk(i, carry):
            pltpu.sync_copy(sd_hbm.at[chbase + i], sd)
            pltpu.async_copy(x_hbm.at[sd.at[0]], rows, sem).wait()
            pltpu.sync_copy(rows, acc_sum.at[sd.at[1]], add=True)
            return carry

        lax.fori_loop(0, nch, chunk, 0)
        plsc.subcore_barrier()
        sl = pl.ds(s * ZROWS, ZROWS)
        pltpu.sync_copy(acc_sum.at[sl], sum_out.at[c, sl])

    return pl.kernel(
        body_nocnt,
        out_type=tuple(out_type),
        mesh=mesh,
        scratch_types=tuple(scratch),
        name="sc_segsum",
    )


_sc_scatter = _make_sc_scatter(False)

_HBLK = 2000


def _cnt_hist(dst_col):
    def body(d_ref, o_ref):
        i = pl.program_id(0)
        d = d_ref[...]
        lane = lax.broadcasted_iota(jnp.int32, (1, 128), 1)
        a = ((d >> 7) == lane).astype(jnp.bfloat16)
        b = ((d & 127) == lane).astype(jnp.bfloat16)
        blk = lax.dot_general(a, b, (((0,), (0,)), ((), ())),
                              preferred_element_type=jnp.float32)

        @pl.when(i == 0)
        def _():
            o_ref[...] = blk

        @pl.when(i != 0)
        def _():
            o_ref[...] += blk

    return pl.pallas_call(
        body,
        grid=(E // _HBLK,),
        in_specs=[pl.BlockSpec((_HBLK, 1), lambda i: (i, 0))],
        out_specs=pl.BlockSpec((128, 128), lambda i: (0, 0)),
        out_shape=jax.ShapeDtypeStruct((128, 128), jnp.float32),
    )(dst_col)

_BM = 2048


def _sage_dense(p, cnt, xin, Wl, Wr, b, relu):
    def body(p_ref, c_ref, x_ref, wl_ref, wr_ref, b_ref, o_ref):
        psum = p_ref[0] + p_ref[1]
        csum = c_ref[...]
        agg = psum / jnp.maximum(csum, 1.0)
        acc = lax.dot_general(agg, wl_ref[...], (((1,), (1,)), ((), ())),
                              preferred_element_type=jnp.float32)
        acc = acc + lax.dot_general(x_ref[...], wr_ref[...],
                                    (((1,), (1,)), ((), ())),
                                    preferred_element_type=jnp.float32)
        acc = acc + b_ref[...]
        if relu:
            acc = jnp.maximum(acc, 0.0)
        o_ref[...] = acc

    return pl.pallas_call(
        body,
        grid=(NP // _BM,),
        in_specs=[
            pl.BlockSpec((NC, _BM, D), lambda i: (0, i, 0)),
            pl.BlockSpec((_BM, 1), lambda i: (i, 0)),
            pl.BlockSpec((_BM, D), lambda i: (i, 0)),
            pl.BlockSpec((D, D), lambda i: (0, 0)),
            pl.BlockSpec((D, D), lambda i: (0, 0)),
            pl.BlockSpec((1, D), lambda i: (0, 0)),
        ],
        out_specs=pl.BlockSpec((_BM, D), lambda i: (i, 0)),
        out_shape=jax.ShapeDtypeStruct((NP, D), jnp.float32),
    )(p, cnt, xin, Wl, Wr, b)


_DBM = 1024


def _decoder(h2):
    def body(a_ref, b_ref, o_ref):
        o_ref[...] = lax.dot_general(a_ref[...], b_ref[...],
                                     (((1,), (1,)), ((), ())),
                                     preferred_element_type=jnp.float32)

    drug = lax.slice(h2, (NUM_RNA, 0), (N, D))
    nb = -(-NUM_RNA // _DBM)
    return pl.pallas_call(
        body,
        grid=(nb, nb),
        in_specs=[
            pl.BlockSpec((_DBM, D), lambda i, j: (i, 0)),
            pl.BlockSpec((_DBM, D), lambda i, j: (j, 0)),
        ],
        out_specs=pl.BlockSpec((_DBM, _DBM), lambda i, j: (i, j)),
        out_shape=jax.ShapeDtypeStruct((NUM_RNA, NUM_RNA), jnp.float32),
    )(h2, drug)


def _layout_edges(arr, fillval):
    parts = []
    off = 0
    for nreal, nchunk in ((R0, C0),) * NS + ((R1, C1),) * NS:
        parts.append(lax.slice(arr, (off,), (off + nreal,)))
        npad = nchunk * CHUNK - nreal
        parts.append(jnp.full((npad,), fillval, jnp.int32))
        off += nreal
    return jnp.concatenate(parts)


def kernel(x, edge_index, W1_l, W1_r, b1, W2_l, W2_r, b2):
    src = edge_index[0]
    dst = edge_index[1]
    src_p = _layout_edges(src, 0)
    dst_p = _layout_edges(dst, TRASH)
    sd = jnp.concatenate([src_p.reshape(NCHUNKS, 1, CHUNK),
                          dst_p.reshape(NCHUNKS, 1, CHUNK)], axis=1)
    x_p = jnp.concatenate([x, jnp.zeros((NP - N, D), jnp.float32)], axis=0)
    zsum = jnp.zeros((ZROWS, D), jnp.float32)
    b1r = b1.reshape(1, D)
    b2r = b2.reshape(1, D)

    cnt128 = _cnt_hist(dst.reshape(E, 1))
    c1 = cnt128.reshape(128 * 128, 1)[:NP]
    (p1,) = _sc_scatter(x_p, sd, zsum)
    h = _sage_dense(p1, c1, x_p, W1_l, W1_r, b1r, relu=True)
    (p2,) = _sc_scatter(h, sd, zsum)
    h2 = _sage_dense(p2, c1, h, W2_l, W2_r, b2r, relu=False)
    return _decoder(h2)


# speedup vs baseline: 1.2441x; 1.2441x over previous
import functools

import jax
import jax.numpy as jnp
from jax import lax
from jax.experimental import pallas as pl
from jax.experimental.pallas import tpu as pltpu
from jax.experimental.pallas import tpu_sc as plsc

N = 10000
NUM_RNA = 5000
E = 320000
D = 128

NC = 2
NS = 16
NW = NC * NS
CHUNK = 128
CHUNKS_PER_W = -(-E // (NW * CHUNK))
EPW = CHUNKS_PER_W * CHUNK
EP = EPW * NW
TRASH = N
NP = 10240
ZROWS = NP // NS


def _make_sc_scatter(with_cnt):
    mesh = plsc.VectorSubcoreMesh(core_axis_name="c", subcore_axis_name="s")
    out_type = [jax.ShapeDtypeStruct((NC, NP, D), jnp.float32)]
    scratch = [
        pltpu.VMEM_SHARED((NP, D), jnp.float32),
        pltpu.VMEM((2, CHUNK), jnp.int32),
        pltpu.VMEM((CHUNK, D), jnp.float32),
        pltpu.SemaphoreType.DMA,
    ]

    def body_nocnt(x_hbm, sd_hbm, zsum_hbm,
                   sum_out, acc_sum, sd, rows, sem):
        c = lax.axis_index("c")
        s = lax.axis_index("s")
        wid = c * NS + s
        pltpu.sync_copy(zsum_hbm, acc_sum.at[pl.ds(s * ZROWS, ZROWS)])
        plsc.subcore_barrier()

        def chunk(i, carry):
            pltpu.sync_copy(sd_hbm.at[wid * CHUNKS_PER_W + i], sd)
            pltpu.async_copy(x_hbm.at[sd.at[0]], rows, sem).wait()
            pltpu.sync_copy(rows, acc_sum.at[sd.at[1]], add=True)
            return carry

        lax.fori_loop(0, CHUNKS_PER_W, chunk, 0)
        plsc.subcore_barrier()
        sl = pl.ds(s * ZROWS, ZROWS)
        pltpu.sync_copy(acc_sum.at[sl], sum_out.at[c, sl])

    return pl.kernel(
        body_nocnt,
        out_type=tuple(out_type),
        mesh=mesh,
        scratch_types=tuple(scratch),
        name="sc_segsum",
    )


_sc_scatter = _make_sc_scatter(False)

_HBLK = 2000


def _cnt_hist(dst_col):
    def body(d_ref, o_ref):
        i = pl.program_id(0)
        d = d_ref[...]
        lane = lax.broadcasted_iota(jnp.int32, (1, 128), 1)
        a = ((d >> 7) == lane).astype(jnp.bfloat16)
        b = ((d & 127) == lane).astype(jnp.bfloat16)
        blk = lax.dot_general(a, b, (((0,), (0,)), ((), ())),
                              preferred_element_type=jnp.float32)

        @pl.when(i == 0)
        def _():
            o_ref[...] = blk

        @pl.when(i != 0)
        def _():
            o_ref[...] += blk

    return pl.pallas_call(
        body,
        grid=(E // _HBLK,),
        in_specs=[pl.BlockSpec((_HBLK, 1), lambda i: (i, 0))],
        out_specs=pl.BlockSpec((128, 128), lambda i: (0, 0)),
        out_shape=jax.ShapeDtypeStruct((128, 128), jnp.float32),
    )(dst_col)

_BM = 2048


def _sage_dense(p, cnt, xin, Wl, Wr, b, relu):
    def body(p_ref, c_ref, x_ref, wl_ref, wr_ref, b_ref, o_ref):
        psum = p_ref[0] + p_ref[1]
        csum = c_ref[...]
        agg = psum / jnp.maximum(csum, 1.0)
        acc = lax.dot_general(agg, wl_ref[...], (((1,), (1,)), ((), ())),
                              preferred_element_type=jnp.float32)
        acc = acc + lax.dot_general(x_ref[...], wr_ref[...],
                                    (((1,), (1,)), ((), ())),
                                    preferred_element_type=jnp.float32)
        acc = acc + b_ref[...]
        if relu:
            acc = jnp.maximum(acc, 0.0)
        o_ref[...] = acc

    return pl.pallas_call(
        body,
        grid=(NP // _BM,),
        in_specs=[
            pl.BlockSpec((NC, _BM, D), lambda i: (0, i, 0)),
            pl.BlockSpec((_BM, 1), lambda i: (i, 0)),
            pl.BlockSpec((_BM, D), lambda i: (i, 0)),
            pl.BlockSpec((D, D), lambda i: (0, 0)),
            pl.BlockSpec((D, D), lambda i: (0, 0)),
            pl.BlockSpec((1, D), lambda i: (0, 0)),
        ],
        out_specs=pl.BlockSpec((_BM, D), lambda i: (i, 0)),
        out_shape=jax.ShapeDtypeStruct((NP, D), jnp.float32),
    )(p, cnt, xin, Wl, Wr, b)


_DBM = 1024


def _decoder(h2):
    def body(a_ref, b_ref, o_ref):
        o_ref[...] = lax.dot_general(a_ref[...], b_ref[...],
                                     (((1,), (1,)), ((), ())),
                                     preferred_element_type=jnp.float32)

    drug = lax.slice(h2, (NUM_RNA, 0), (N, D))
    nb = -(-NUM_RNA // _DBM)
    return pl.pallas_call(
        body,
        grid=(nb, nb),
        in_specs=[
            pl.BlockSpec((_DBM, D), lambda i, j: (i, 0)),
            pl.BlockSpec((_DBM, D), lambda i, j: (j, 0)),
        ],
        out_specs=pl.BlockSpec((_DBM, _DBM), lambda i, j: (i, j)),
        out_shape=jax.ShapeDtypeStruct((NUM_RNA, NUM_RNA), jnp.float32),
    )(h2, drug)


def kernel(x, edge_index, W1_l, W1_r, b1, W2_l, W2_r, b2):
    src = edge_index[0]
    dst = edge_index[1]
    pad = EP - E
    src_p = jnp.concatenate([src, jnp.zeros((pad,), jnp.int32)])
    dst_p = jnp.concatenate([dst, jnp.full((pad,), TRASH, jnp.int32)])
    x_p = jnp.concatenate([x, jnp.zeros((NP - N, D), jnp.float32)], axis=0)
    zsum = jnp.zeros((ZROWS, D), jnp.float32)
    b1r = b1.reshape(1, D)
    b2r = b2.reshape(1, D)

    nchunks = EP // CHUNK
    sd = jnp.concatenate([src_p.reshape(nchunks, 1, CHUNK),
                          dst_p.reshape(nchunks, 1, CHUNK)], axis=1)
    cnt128 = _cnt_hist(dst.reshape(E, 1))
    c1 = cnt128.reshape(128 * 128, 1)[:NP]
    (p1,) = _sc_scatter(x_p, sd, zsum)
    h = _sage_dense(p1, c1, x_p, W1_l, W1_r, b1r, relu=True)
    (p2,) = _sc_scatter(h, sd, zsum)
    h2 = _sage_dense(p2, c1, h, W2_l, W2_r, b2r, relu=False)
    return _decoder(h2)


# speedup vs baseline: 1.3482x; 1.0837x over previous
import functools

import jax
import jax.numpy as jnp
from jax import lax
from jax.experimental import pallas as pl
from jax.experimental.pallas import tpu as pltpu
from jax.experimental.pallas import tpu_sc as plsc

N = 10000
NUM_RNA = 5000
E = 320000
D = 128

NC = 2
NS = 16
NW = NC * NS
CHUNK = 128
CHUNKS_PER_W = -(-E // (NW * CHUNK))
EPW = CHUNKS_PER_W * CHUNK
EP = EPW * NW
TRASH = N
NP = 10240
ZROWS = NP // NS


def _make_sc_scatter(with_cnt):
    mesh = plsc.VectorSubcoreMesh(core_axis_name="c", subcore_axis_name="s")
    out_type = [jax.ShapeDtypeStruct((NC, NP, D), jnp.float32)]
    scratch = [
        pltpu.VMEM_SHARED((NP, D), jnp.float32),
        [pltpu.VMEM((2, CHUNK), jnp.int32) for _ in range(2)],
        pltpu.VMEM((CHUNK, D), jnp.float32),
        pltpu.SemaphoreType.DMA,
        [pltpu.SemaphoreType.DMA for _ in range(2)],
    ]

    def body_nocnt(x_hbm, sd_hbm, zsum_hbm,
                   sum_out, acc_sum, sd, rows, sem, isems):
        c = lax.axis_index("c")
        s = lax.axis_index("s")
        wid = c * NS + s
        cb = wid * CHUNKS_PER_W
        pltpu.sync_copy(zsum_hbm, acc_sum.at[pl.ds(s * ZROWS, ZROWS)])
        plsc.subcore_barrier()

        def do(b, ch):
            pltpu.make_async_copy(sd_hbm.at[ch], sd[b], isems[b]).wait()
            pltpu.async_copy(x_hbm.at[sd[b].at[0]], rows, sem).wait()
            pltpu.sync_copy(rows, acc_sum.at[sd[b].at[1]], add=True)

        pltpu.async_copy(sd_hbm.at[cb], sd[0], isems[0])

        def pair(j, carry):
            pltpu.async_copy(sd_hbm.at[cb + 2 * j + 1], sd[1], isems[1])
            do(0, cb + 2 * j)
            pltpu.async_copy(sd_hbm.at[cb + 2 * j + 2], sd[0], isems[0])
            do(1, cb + 2 * j + 1)
            return carry

        lax.fori_loop(0, (CHUNKS_PER_W - 1) // 2, pair, 0)
        do(0, cb + CHUNKS_PER_W - 1)
        plsc.subcore_barrier()
        sl = pl.ds(s * ZROWS, ZROWS)
        pltpu.sync_copy(acc_sum.at[sl], sum_out.at[c, sl])

    return pl.kernel(
        body_nocnt,
        out_type=tuple(out_type),
        mesh=mesh,
        scratch_types=tuple(scratch),
        name="sc_segsum",
    )


_sc_scatter = _make_sc_scatter(False)

_HBLK = 2000


def _cnt_hist(dst_col):
    def body(d_ref, o_ref):
        i = pl.program_id(0)
        d = d_ref[...]
        lane = lax.broadcasted_iota(jnp.int32, (1, 128), 1)
        a = ((d >> 7) == lane).astype(jnp.bfloat16)
        b = ((d & 127) == lane).astype(jnp.bfloat16)
        blk = lax.dot_general(a, b, (((0,), (0,)), ((), ())),
                              preferred_element_type=jnp.float32)

        @pl.when(i == 0)
        def _():
            o_ref[...] = blk

        @pl.when(i != 0)
        def _():
            o_ref[...] += blk

    return pl.pallas_call(
        body,
        grid=(E // _HBLK,),
        in_specs=[pl.BlockSpec((_HBLK, 1), lambda i: (i, 0))],
        out_specs=pl.BlockSpec((128, 128), lambda i: (0, 0)),
        out_shape=jax.ShapeDtypeStruct((128, 128), jnp.float32),
    )(dst_col)

_BM = 2048


def _sage_dense(p, cnt, xin, Wl, Wr, b, relu):
    def body(p_ref, c_ref, x_ref, wl_ref, wr_ref, b_ref, o_ref):
        psum = p_ref[0] + p_ref[1]
        csum = c_ref[...]
        agg = psum / jnp.maximum(csum, 1.0)
        acc = lax.dot_general(agg, wl_ref[...], (((1,), (1,)), ((), ())),
                              preferred_element_type=jnp.float32)
        acc = acc + lax.dot_general(x_ref[...], wr_ref[...],
                                    (((1,), (1,)), ((), ())),
                                    preferred_element_type=jnp.float32)
        acc = acc + b_ref[...]
        if relu:
            acc = jnp.maximum(acc, 0.0)
        o_ref[...] = acc

    return pl.pallas_call(
        body,
        grid=(NP // _BM,),
        in_specs=[
            pl.BlockSpec((NC, _BM, D), lambda i: (0, i, 0)),
            pl.BlockSpec((_BM, 1), lambda i: (i, 0)),
            pl.BlockSpec((_BM, D), lambda i: (i, 0)),
            pl.BlockSpec((D, D), lambda i: (0, 0)),
            pl.BlockSpec((D, D), lambda i: (0, 0)),
            pl.BlockSpec((1, D), lambda i: (0, 0)),
        ],
        out_specs=pl.BlockSpec((_BM, D), lambda i: (i, 0)),
        out_shape=jax.ShapeDtypeStruct((NP, D), jnp.float32),
    )(p, cnt, xin, Wl, Wr, b)


_DBM = 1024


def _decoder(h2):
    def body(a_ref, b_ref, o_ref):
        o_ref[...] = lax.dot_general(a_ref[...], b_ref[...],
                                     (((1,), (1,)), ((), ())),
                                     preferred_element_type=jnp.float32)

    drug = lax.slice(h2, (NUM_RNA, 0), (N, D))
    nb = -(-NUM_RNA // _DBM)
    return pl.pallas_call(
        body,
        grid=(nb, nb),
        in_specs=[
            pl.BlockSpec((_DBM, D), lambda i, j: (i, 0)),
            pl.BlockSpec((_DBM, D), lambda i, j: (j, 0)),
        ],
        out_specs=pl.BlockSpec((_DBM, _DBM), lambda i, j: (i, j)),
        out_shape=jax.ShapeDtypeStruct((NUM_RNA, NUM_RNA), jnp.float32),
    )(h2, drug)


def kernel(x, edge_index, W1_l, W1_r, b1, W2_l, W2_r, b2):
    src = edge_index[0]
    dst = edge_index[1]
    pad = EP - E
    src_p = jnp.concatenate([src, jnp.zeros((pad,), jnp.int32)])
    dst_p = jnp.concatenate([dst, jnp.full((pad,), TRASH, jnp.int32)])
    x_p = jnp.concatenate([x, jnp.zeros((NP - N, D), jnp.float32)], axis=0)
    zsum = jnp.zeros((ZROWS, D), jnp.float32)
    b1r = b1.reshape(1, D)
    b2r = b2.reshape(1, D)

    nchunks = EP // CHUNK
    sd = jnp.concatenate([src_p.reshape(nchunks, 1, CHUNK),
                          dst_p.reshape(nchunks, 1, CHUNK)], axis=1)
    cnt128 = _cnt_hist(dst.reshape(E, 1))
    c1 = cnt128.reshape(128 * 128, 1)[:NP]
    (p1,) = _sc_scatter(x_p, sd, zsum)
    h = _sage_dense(p1, c1, x_p, W1_l, W1_r, b1r, relu=True)
    (p2,) = _sc_scatter(h, sd, zsum)
    h2 = _sage_dense(p2, c1, h, W2_l, W2_r, b2r, relu=False)
    return _decoder(h2)
